# Initial kernel scaffold; baseline (speedup 1.0000x reference)
import jax
import jax.numpy as jnp
from jax.experimental import pallas as pl


def kernel(x, Wg, W1, b1, W2, b2):
    raise NotImplementedError("write your pallas kernel here")


# speedup vs baseline: 4.6283x; 4.6283x over previous
import functools

import jax
import jax.numpy as jnp
from jax.experimental import pallas as pl
from jax.experimental.pallas import tpu as pltpu

B, S, H, E, TOP_K = 1, 2048, 1024, 8, 2
F = 4 * H

BM = 256
FCC = 2048
FC = F // FCC
NB = (S * TOP_K) // BM + E
N_SLOTS = NB * BM


def _moe_block_kernel(eids_ref, rows_ref, nact_ref,
                      x_ref, w1_ref, b1_ref, w2_ref, b2_ref, wmat_ref,
                      out_ref, xg_ref, yacc_ref):
    b = pl.program_id(0)
    fc = pl.program_id(1)

    @pl.when((b == 0) & (fc == 0))
    def _init_out():
        out_ref[...] = jnp.zeros_like(out_ref)

    @pl.when(b < nact_ref[0])
    def _active():
        @pl.when(fc == 0)
        def _gather():
            def body(i, _):
                r = rows_ref[b * BM + i]
                xg_ref[pl.ds(i, 1), :] = x_ref[pl.ds(r, 1), :]
                return 0
            jax.lax.fori_loop(0, BM, body, 0, unroll=8)

        h = jax.lax.dot_general(
            xg_ref[...], w1_ref[0],
            (((1,), (1,)), ((), ())), preferred_element_type=jnp.float32)
        h = h + b1_ref[0, 0]
        h = 0.5 * h * (1.0 + jax.lax.erf(h * 0.7071067811865476))
        y = jax.lax.dot_general(
            h, w2_ref[0],
            (((1,), (1,)), ((), ())), preferred_element_type=jnp.float32)

        @pl.when(fc == 0)
        def _first():
            yacc_ref[...] = y + b2_ref[0]

        @pl.when(fc > 0)
        def _rest():
            yacc_ref[...] = yacc_ref[...] + y

        @pl.when(fc == FC - 1)
        def _scatter():
            yacc_ref[...] = yacc_ref[...] * wmat_ref[0][:, 0:1]

            def body(i, _):
                r = rows_ref[b * BM + i]
                out_ref[pl.ds(r, 1), :] += yacc_ref[pl.ds(i, 1), :]
                return 0
            jax.lax.fori_loop(0, BM, body, 0, unroll=8)


@functools.partial(jax.jit, static_argnums=())
def kernel(x, Wg, W1, b1, W2, b2):
    logits = jnp.einsum('bsh,eh->bse', jax.lax.stop_gradient(x), Wg)
    probs = jax.nn.softmax(logits, axis=-1)
    top_k_weights, top_k_indices = jax.lax.top_k(probs, TOP_K)

    e_flat = top_k_indices.reshape(S * TOP_K).astype(jnp.int32)
    w_flat = top_k_weights.reshape(S * TOP_K)
    onehot = (e_flat[:, None] == jnp.arange(E, dtype=jnp.int32)[None, :])
    counts = onehot.sum(axis=0, dtype=jnp.int32)
    bpe = (counts + BM - 1) // BM
    block_start = jnp.concatenate(
        [jnp.zeros((1,), jnp.int32), jnp.cumsum(bpe)[:-1].astype(jnp.int32)])
    nactive = (block_start[E - 1] + bpe[E - 1]).astype(jnp.int32)
    slot_start = block_start * BM
    rank = jnp.cumsum(onehot.astype(jnp.int32), axis=0) - 1
    rank_e = jnp.take_along_axis(rank, e_flat[:, None], axis=1)[:, 0]
    pos = slot_start[e_flat] + rank_e
    tok = jnp.arange(S * TOP_K, dtype=jnp.int32) // TOP_K
    rows = jnp.zeros((N_SLOTS,), jnp.int32).at[pos].set(tok)
    wvals = jnp.zeros((N_SLOTS,), jnp.float32).at[pos].set(w_flat)
    wmat = jnp.broadcast_to(
        wvals.reshape(NB, BM)[:, :, None], (NB, BM, 128))
    eids = (jnp.arange(NB, dtype=jnp.int32)[:, None]
            >= block_start[None, 1:]).sum(axis=1).astype(jnp.int32)

    x2 = x.reshape(S, H)
    b1r = b1.reshape(E, FC, 1, FCC)
    b2r = b2.reshape(E, 1, H)

    grid_spec = pltpu.PrefetchScalarGridSpec(
        num_scalar_prefetch=3,
        grid=(NB, FC),
        in_specs=[
            pl.BlockSpec((S, H), lambda b, fc, eids, rows, nact: (0, 0)),
            pl.BlockSpec((1, FCC, H),
                         lambda b, fc, eids, rows, nact: (eids[b], fc, 0)),
            pl.BlockSpec((1, 1, 1, FCC),
                         lambda b, fc, eids, rows, nact: (eids[b], fc, 0, 0)),
            pl.BlockSpec((1, H, FCC),
                         lambda b, fc, eids, rows, nact: (eids[b], 0, fc)),
            pl.BlockSpec((1, 1, H),
                         lambda b, fc, eids, rows, nact: (eids[b], 0, 0)),
            pl.BlockSpec((1, BM, 128),
                         lambda b, fc, eids, rows, nact: (b, 0, 0)),
        ],
        out_specs=pl.BlockSpec((S, H), lambda b, fc, eids, rows, nact: (0, 0)),
        scratch_shapes=[
            pltpu.VMEM((BM, H), jnp.float32),
            pltpu.VMEM((BM, H), jnp.float32),
        ],
    )

    out = pl.pallas_call(
        _moe_block_kernel,
        grid_spec=grid_spec,
        out_shape=jax.ShapeDtypeStruct((S, H), jnp.float32),
        compiler_params=pltpu.CompilerParams(
            dimension_semantics=("arbitrary", "arbitrary"),
            vmem_limit_bytes=100 * 1024 * 1024,
        ),
    )(eids, rows, nactive.reshape(1), x2, W1, b1r, W2, b2r, wmat)

    return out.reshape(B, S, H)
